# baseline (speedup 1.0000x reference)
import functools

import jax
import jax.numpy as jnp
from jax import lax
from jax.experimental import pallas as pl
from jax.experimental.pallas import tpu as pltpu
from jax.experimental.pallas import tpu_sc as plsc

H = 8
C = 33
D = H * C
DH = 132
DHP = 144
BLK = 256
NP = 10240
N = 10000
ET = 330000
EP = 330240
CHA = 240
CHB = 96
NTILES = 32
ITA = EP // NTILES // CHA
EPT_B = EP // 16
ITB = EPT_B // CHB
RPT = NP // 16


def _elu(x):
    return jnp.where(x > 0, x, jnp.exp(x) - 1.0)


def _lrelu(x):
    return jnp.where(x > 0, x, 0.2 * x)


def _layer0_body(x_ref, w_ref, as_ref, ad_ref,
                 hp0_ref, hp1_ref, es_ref, ed_ref, mg_ref):
    i = pl.program_id(0)
    hp = jnp.dot(x_ref[...], w_ref[...], preferred_element_type=jnp.float32)
    es = jnp.dot(hp, as_ref[...], preferred_element_type=jnp.float32)
    ed = jnp.dot(hp, ad_ref[...], preferred_element_type=jnp.float32)
    es_ref[...] = es
    ed_ref[...] = ed
    z = jnp.zeros((hp.shape[0], DHP - DH), jnp.float32)
    hp0_ref[...] = jnp.concatenate([hp[:, :DH], z], axis=1)
    hp1_ref[...] = jnp.concatenate([hp[:, DH:], z], axis=1)
    cur = jnp.max(es, axis=0, keepdims=True)

    @pl.when(i == 0)
    def _():
        mg_ref[...] = cur

    @pl.when(i > 0)
    def _():
        mg_ref[...] = jnp.maximum(mg_ref[...], cur)


def _layerk_body(r0_ref, r1_ref, bh0_ref, bh1_ref, w0_ref, w1_ref,
                 as_ref, ad_ref,
                 hp0_ref, hp1_ref, es_ref, ed_ref, mg_ref):
    i = pl.program_id(0)
    e0 = _elu(r0_ref[...] + bh0_ref[...])
    e1 = _elu(r1_ref[...] + bh1_ref[...])
    hp = (jnp.dot(e0, w0_ref[...], preferred_element_type=jnp.float32)
          + jnp.dot(e1, w1_ref[...], preferred_element_type=jnp.float32))
    es = jnp.dot(hp, as_ref[...], preferred_element_type=jnp.float32)
    ed = jnp.dot(hp, ad_ref[...], preferred_element_type=jnp.float32)
    es_ref[...] = es
    ed_ref[...] = ed
    z = jnp.zeros((hp.shape[0], DHP - DH), jnp.float32)
    hp0_ref[...] = jnp.concatenate([hp[:, :DH], z], axis=1)
    hp1_ref[...] = jnp.concatenate([hp[:, DH:], z], axis=1)
    cur = jnp.max(es, axis=0, keepdims=True)

    @pl.when(i == 0)
    def _():
        mg_ref[...] = cur

    @pl.when(i > 0)
    def _():
        mg_ref[...] = jnp.maximum(mg_ref[...], cur)


def _head_body(r0_ref, r1_ref, bh0_ref, bh1_ref, w0_ref, w1_ref, hb_ref,
               z_ref):
    e0 = _elu(r0_ref[...] + bh0_ref[...])
    e1 = _elu(r1_ref[...] + bh1_ref[...])
    z_ref[...] = (jnp.dot(e0, w0_ref[...], preferred_element_type=jnp.float32)
                  + jnp.dot(e1, w1_ref[...], preferred_element_type=jnp.float32)
                  + hb_ref[...])


def _node_outs():
    return (
        [jax.ShapeDtypeStruct((NP, DHP), jnp.float32),
         jax.ShapeDtypeStruct((NP, DHP), jnp.float32),
         jax.ShapeDtypeStruct((NP, H), jnp.float32),
         jax.ShapeDtypeStruct((NP, H), jnp.float32),
         jax.ShapeDtypeStruct((1, H), jnp.float32)],
        [pl.BlockSpec((BLK, DHP), lambda i: (i, 0)),
         pl.BlockSpec((BLK, DHP), lambda i: (i, 0)),
         pl.BlockSpec((BLK, H), lambda i: (i, 0)),
         pl.BlockSpec((BLK, H), lambda i: (i, 0)),
         pl.BlockSpec((1, H), lambda i: (0, 0))],
    )


def _full(shape):
    return pl.BlockSpec(shape, lambda i: tuple(0 for _ in shape))


def _tc_layer0(x_pad, W, As, Ad):
    outs, ospecs = _node_outs()
    return pl.pallas_call(
        _layer0_body,
        grid=(NP // BLK,),
        in_specs=[pl.BlockSpec((BLK, 128), lambda i: (i, 0)),
                  _full((128, D)), _full((D, H)), _full((D, H))],
        out_specs=ospecs,
        out_shape=outs,
    )(x_pad, W, As, Ad)


def _tc_layerk(r0, r1, bh0, bh1, Wp0, Wp1, As, Ad):
    outs, ospecs = _node_outs()
    return pl.pallas_call(
        _layerk_body,
        grid=(NP // BLK,),
        in_specs=[pl.BlockSpec((BLK, DHP), lambda i: (i, 0)),
                  pl.BlockSpec((BLK, DHP), lambda i: (i, 0)),
                  _full((1, DHP)), _full((1, DHP)),
                  _full((DHP, D)), _full((DHP, D)),
                  _full((D, H)), _full((D, H))],
        out_specs=ospecs,
        out_shape=outs,
    )(r0, r1, bh0, bh1, Wp0, Wp1, As, Ad)


def _tc_head(r0, r1, bh0, bh1, hw0, hw1, hb):
    HB = 200
    return pl.pallas_call(
        _head_body,
        grid=(N // HB,),
        in_specs=[pl.BlockSpec((HB, DHP), lambda i: (i, 0)),
                  pl.BlockSpec((HB, DHP), lambda i: (i, 0)),
                  _full((1, DHP)), _full((1, DHP)),
                  _full((DHP, DH)), _full((DHP, DH)),
                  _full((1, DH))],
        out_specs=pl.BlockSpec((HB, DH), lambda i: (i, 0)),
        out_shape=jax.ShapeDtypeStruct((N, DH), jnp.float32),
    )(r0, r1, bh0, bh1, hw0, hw1, hb)


@functools.lru_cache(maxsize=None)
def _sc_mesh():
    return plsc.VectorSubcoreMesh(core_axis_name="c", subcore_axis_name="s",
                                  num_cores=2, num_subcores=16)


_SC_PARAMS = pltpu.CompilerParams(needs_layout_passes=False,
                                  use_tc_tiling_on_sc=False)


def _iota16():
    return lax.iota(jnp.int32, 16)


def _g16(ref2d, k):
    i = _iota16()
    return plsc.load_gather(ref2d, [(i >> 3) + 2 * k, i & 7])


def _s16(ref2d, k, x):
    i = _iota16()
    plsc.store_scatter(ref2d, [(i >> 3) + 2 * k, i & 7], x)


def _lrelu16(x):
    return jnp.where(x > 0, x, 0.2 * x)


@functools.lru_cache(maxsize=None)
def _make_pass_a():
    buf = lambda shape, dt: [pltpu.VMEM(shape, dt), pltpu.VMEM(shape, dt)]
    return pl.kernel(
        _sc_pass_a,
        out_type=[jax.ShapeDtypeStruct((EP, H), jnp.float32),
                  jax.ShapeDtypeStruct((NP, H), jnp.float32),
                  jax.ShapeDtypeStruct((NP, H), jnp.float32)],
        mesh=_sc_mesh(),
        compiler_params=_SC_PARAMS,
        scratch_types=[buf((CHA,), jnp.int32),
                       buf((CHA,), jnp.int32),
                       buf((CHA, H), jnp.float32),
                       buf((CHA, H), jnp.float32),
                       pltpu.VMEM((CHA, H), jnp.float32),
                       pltpu.VMEM((16,), jnp.float32),
                       pltpu.VMEM((RPT, H), jnp.float32),
                       pltpu.VMEM_SHARED((NP, H), jnp.float32),
                       [pltpu.SemaphoreType.DMA, pltpu.SemaphoreType.DMA],
                       [pltpu.SemaphoreType.DMA, pltpu.SemaphoreType.DMA]],
    )


def _sc_pass_a(src_hbm, dst_hbm, es_hbm, ed_hbm, mg_hbm,
               ex_hbm, s0_hbm, s1_hbm,
               src_v, dst_v, esg, edg, exv, mgv, zbuf, s_sh, sem1, sem2):
    c = lax.axis_index("c")
    s = lax.axis_index("s")
    base = (c * 16 + s) * (EP // NTILES)

    def zb(k, _):
        _s16(zbuf, k, jnp.zeros((16,), jnp.float32))
        return 0
    lax.fori_loop(0, RPT * H // 16, zb, 0)
    pltpu.sync_copy(zbuf, s_sh.at[pl.ds(s * RPT, RPT)])
    pltpu.sync_copy(mg_hbm, mgv)
    plsc.subcore_barrier()

    mgvec = mgv[...]

    def start(i, b):
        off = base + i * CHA
        pltpu.sync_copy(src_hbm.at[pl.ds(off, CHA)], src_v[b])
        pltpu.sync_copy(dst_hbm.at[pl.ds(off, CHA)], dst_v[b])
        pltpu.async_copy(es_hbm.at[src_v[b]], esg[b], sem1[b])
        pltpu.async_copy(ed_hbm.at[dst_v[b]], edg[b], sem2[b])

    def process(i, b):
        off = base + i * CHA
        pltpu.make_async_copy(es_hbm.at[src_v[b]], esg[b], sem1[b]).wait()
        pltpu.make_async_copy(ed_hbm.at[dst_v[b]], edg[b], sem2[b]).wait()

        @plsc.parallel_loop(0, CHA // 2, step=1, unroll=4)
        def inner(k):
            esv = _g16(esg[b], k)
            edv = _g16(edg[b], k)
            e = _lrelu16(esv + edv)
            bb = _lrelu16(mgvec + edv)
            _s16(exv, k, jnp.exp(e - bb))
        pltpu.sync_copy(exv, ex_hbm.at[pl.ds(off, CHA)])
        pltpu.sync_copy(exv, s_sh.at[dst_v[b]], add=True)

    start(0, 0)

    def pair(g, _):
        start(2 * g + 1, 1)
        process(2 * g, 0)
        start(2 * g + 2, 0)
        process(2 * g + 1, 1)
        return 0
    lax.fori_loop(0, ITA // 2, pair, 0)
    process(ITA - 1, 0)

    plsc.subcore_barrier()
    rs = pl.ds(s * RPT, RPT)

    @pl.when(c == 0)
    def _():
        pltpu.sync_copy(s_sh.at[rs], s0_hbm.at[rs])

    @pl.when(c == 1)
    def _():
        pltpu.sync_copy(s_sh.at[rs], s1_hbm.at[rs])


@functools.lru_cache(maxsize=None)
def _make_pass_b():
    buf = lambda shape, dt: [pltpu.VMEM(shape, dt), pltpu.VMEM(shape, dt)]
    sem2 = lambda: [pltpu.SemaphoreType.DMA, pltpu.SemaphoreType.DMA]
    return pl.kernel(
        _sc_pass_b,
        out_type=[jax.ShapeDtypeStruct((NP, DHP), jnp.float32),
                  jax.ShapeDtypeStruct((NP, DHP), jnp.float32)],
        mesh=_sc_mesh(),
        compiler_params=_SC_PARAMS,
        scratch_types=[buf((CHB,), jnp.int32),
                       buf((CHB,), jnp.int32),
                       buf((CHB, DHP), jnp.float32),
                       buf((CHB, H), jnp.float32),
                       buf((CHB, H), jnp.float32),
                       buf((CHB, H), jnp.float32),
                       pltpu.VMEM((CHB * H,), jnp.float32),
                       pltpu.VMEM_SHARED((NP, DHP), jnp.float32),
                       sem2(), sem2(), sem2(), sem2()],
    )


def _sc_pass_b(src_hbm, dst_hbm, ex_hbm, s0_hbm, s1_hbm, hp0_hbm, hp1_hbm,
               r0_hbm, r1_hbm,
               src_v, dst_v, rows_v, exv, s0g, s1g, af, out_sh,
               sem1, sem2, sem3, sem4):
    c = lax.axis_index("c")
    s = lax.axis_index("s")

    def half(hp_hbm, r_hbm, head_base):
        i16 = _iota16()
        hms = []
        for j in range(DHP // 16):
            pos = i16 + j * 16
            hms.append(jnp.minimum(pos // C, H // 2 - 1) + head_base)

        def zrow(e, _):
            for j in range(DHP // 16):
                plsc.store_scatter(
                    rows_v[0], [jnp.broadcast_to(e, (16,)), i16 + j * 16],
                    jnp.zeros((16,), jnp.float32))
            return 0
        lax.fori_loop(0, CHB, zrow, 0)
        for t in range(RPT // CHB):
            pltpu.sync_copy(rows_v[0], out_sh.at[pl.ds(s * RPT + t * CHB, CHB)])
        rem = RPT % CHB
        if rem:
            pltpu.sync_copy(
                rows_v[0].at[pl.ds(0, rem)],
                out_sh.at[pl.ds(s * RPT + (RPT // CHB) * CHB, rem)])
        plsc.subcore_barrier()

        def start(i, b):
            off = s * EPT_B + i * CHB
            pltpu.sync_copy(src_hbm.at[pl.ds(off, CHB)], src_v[b])
            pltpu.sync_copy(dst_hbm.at[pl.ds(off, CHB)], dst_v[b])
            pltpu.async_copy(hp_hbm.at[src_v[b]], rows_v[b], sem1[b])
            pltpu.async_copy(ex_hbm.at[pl.ds(off, CHB)], exv[b], sem2[b])
            pltpu.async_copy(s0_hbm.at[dst_v[b]], s0g[b], sem3[b])
            pltpu.async_copy(s1_hbm.at[dst_v[b]], s1g[b], sem4[b])

        def process(i, b):
            off = s * EPT_B + i * CHB
            pltpu.make_async_copy(ex_hbm.at[pl.ds(off, CHB)], exv[b],
                                  sem2[b]).wait()
            pltpu.make_async_copy(s0_hbm.at[dst_v[b]], s0g[b], sem3[b]).wait()
            pltpu.make_async_copy(s1_hbm.at[dst_v[b]], s1g[b], sem4[b]).wait()

            @plsc.parallel_loop(0, CHB // 2, step=1, unroll=4)
            def aloop(k):
                ex16 = _g16(exv[b], k)
                sv = _g16(s0g[b], k) + _g16(s1g[b], k)
                plsc.store_scatter(af, [i16 + 16 * k], ex16 / (sv + 1e-16))
            pltpu.make_async_copy(hp_hbm.at[src_v[b]], rows_v[b],
                                  sem1[b]).wait()

            @plsc.parallel_loop(0, CHB, step=1, unroll=4)
            def scale(e):
                erow = jnp.broadcast_to(e, (16,))
                e8 = jnp.broadcast_to(e * H, (16,))
                for j in range(DHP // 16):
                    cols = i16 + j * 16
                    a = plsc.load_gather(af, [e8 + hms[j]])
                    v = plsc.load_gather(rows_v[b], [erow, cols])
                    plsc.store_scatter(rows_v[b], [erow, cols], v * a)
            pltpu.sync_copy(rows_v[b], out_sh.at[dst_v[b]], add=True)

        start(0, 0)

        def pair(g, _):
            start(2 * g + 1, 1)
            process(2 * g, 0)
            start(2 * g + 2, 0)
            process(2 * g + 1, 1)
            return 0
        lax.fori_loop(0, ITB // 2, pair, 0)
        process(ITB - 1, 0)

        plsc.subcore_barrier()
        rs = pl.ds(s * RPT, RPT)
        pltpu.sync_copy(out_sh.at[rs], r_hbm.at[rs])

    @pl.when(c == 0)
    def _():
        half(hp0_hbm, r0_hbm, 0)

    @pl.when(c == 1)
    def _():
        half(hp1_hbm, r1_hbm, H // 2)


def _split_pad_W(W):
    Wp0 = jnp.pad(W[:DH], ((0, DHP - DH), (0, 0)))
    Wp1 = jnp.pad(W[DH:], ((0, DHP - DH), (0, 0)))
    return Wp0, Wp1


def _att_mat(a):
    rows = jnp.arange(D)
    return jnp.zeros((D, H), jnp.float32).at[rows, rows // C].set(a.reshape(-1))


def _bias_halves(b):
    bh0 = jnp.pad(b[:DH], (0, DHP - DH)).reshape(1, DHP)
    bh1 = jnp.pad(b[DH:], (0, DHP - DH)).reshape(1, DHP)
    return bh0, bh1


def kernel(x, edge_index, W0, as0, ad0, b0, W1, as1, ad1, b1,
           W2, as2, ad2, b2, W3, as3, ad3, b3, head_W, head_b):
    loop = jnp.arange(N, dtype=edge_index.dtype)
    pad = jnp.full((EP - ET,), N, dtype=edge_index.dtype)
    srcT = jnp.concatenate([edge_index[0], loop, pad])
    dstT = jnp.concatenate([edge_index[1], loop, pad])

    x_pad = jnp.pad(x, ((0, NP - N), (0, 0)))
    params = [(W0, as0, ad0, b0), (W1, as1, ad1, b1),
              (W2, as2, ad2, b2), (W3, as3, ad3, b3)]

    pass_a = _make_pass_a()
    pass_b = _make_pass_b()

    hp0, hp1, es, ed, mg = _tc_layer0(x_pad, W0, _att_mat(as0), _att_mat(ad0))
    for k in (1, 2, 3):
        mg16 = jnp.concatenate([mg[0], mg[0]])
        ex, s0, s1 = pass_a(srcT, dstT, es, ed, mg16)
        r0, r1 = pass_b(srcT, dstT, ex, s0, s1, hp0, hp1)
        W, a_s, a_d, b_prev = params[k][0], params[k][1], params[k][2], params[k - 1][3]
        bh0, bh1 = _bias_halves(b_prev)
        Wp0, Wp1 = _split_pad_W(W)
        hp0, hp1, es, ed, mg = _tc_layerk(r0, r1, bh0, bh1, Wp0, Wp1,
                                          _att_mat(a_s), _att_mat(a_d))
    mg16 = jnp.concatenate([mg[0], mg[0]])
    ex, s0, s1 = pass_a(srcT, dstT, es, ed, mg16)
    r0, r1 = pass_b(srcT, dstT, ex, s0, s1, hp0, hp1)

    bh0, bh1 = _bias_halves(b3)
    hw0 = jnp.pad(head_W[:DH], ((0, DHP - DH), (0, 0)))
    hw1 = jnp.pad(head_W[DH:], ((0, DHP - DH), (0, 0)))
    return _tc_head(r0, r1, bh0, bh1, hw0, hw1, head_b.reshape(1, DH))


# speedup vs baseline: 1.1384x; 1.1384x over previous
import functools

import jax
import jax.numpy as jnp
from jax import lax
from jax.experimental import pallas as pl
from jax.experimental.pallas import tpu as pltpu
from jax.experimental.pallas import tpu_sc as plsc

H = 8
C = 33
D = H * C
DH = 132
DHP = 144
BLK = 256
NP = 10240
N = 10000
ET = 330000
EP = 330240
CHA = 240
CHB = 80
NTILES = 32
ITA = EP // NTILES // CHA
EPT_B = EP // 16
ITB = EPT_B // CHB
RPT = NP // 16
NA = 10016
RPB = NA // 16


def _elu(x):
    return jnp.where(x > 0, x, jnp.exp(x) - 1.0)


def _lrelu(x):
    return jnp.where(x > 0, x, 0.2 * x)


def _layer0_body(x_ref, w_ref, as_ref, ad_ref,
                 hp0_ref, hp1_ref, es_ref, ed_ref, mg_ref):
    i = pl.program_id(0)
    hp = jnp.dot(x_ref[...], w_ref[...], preferred_element_type=jnp.float32)
    es = jnp.dot(hp, as_ref[...], preferred_element_type=jnp.float32)
    ed = jnp.dot(hp, ad_ref[...], preferred_element_type=jnp.float32)
    es_ref[...] = es
    ed_ref[...] = ed
    z = jnp.zeros((hp.shape[0], DHP - DH), jnp.float32)
    hp0_ref[...] = jnp.concatenate([hp[:, :DH], z], axis=1)
    hp1_ref[...] = jnp.concatenate([hp[:, DH:], z], axis=1)
    cur = jnp.max(es, axis=0, keepdims=True)

    @pl.when(i == 0)
    def _():
        mg_ref[...] = cur

    @pl.when(i > 0)
    def _():
        mg_ref[...] = jnp.maximum(mg_ref[...], cur)


def _layerk_body(r0_ref, r1_ref, bh0_ref, bh1_ref, w0_ref, w1_ref,
                 as_ref, ad_ref,
                 hp0_ref, hp1_ref, es_ref, ed_ref, mg_ref):
    i = pl.program_id(0)
    e0 = _elu(r0_ref[...] + bh0_ref[...])
    e1 = _elu(r1_ref[...] + bh1_ref[...])
    hp = (jnp.dot(e0, w0_ref[...], preferred_element_type=jnp.float32)
          + jnp.dot(e1, w1_ref[...], preferred_element_type=jnp.float32))
    es = jnp.dot(hp, as_ref[...], preferred_element_type=jnp.float32)
    ed = jnp.dot(hp, ad_ref[...], preferred_element_type=jnp.float32)
    es_ref[...] = es
    ed_ref[...] = ed
    z = jnp.zeros((hp.shape[0], DHP - DH), jnp.float32)
    hp0_ref[...] = jnp.concatenate([hp[:, :DH], z], axis=1)
    hp1_ref[...] = jnp.concatenate([hp[:, DH:], z], axis=1)
    cur = jnp.max(es, axis=0, keepdims=True)

    @pl.when(i == 0)
    def _():
        mg_ref[...] = cur

    @pl.when(i > 0)
    def _():
        mg_ref[...] = jnp.maximum(mg_ref[...], cur)


def _head_body(r0_ref, r1_ref, bh0_ref, bh1_ref, w0_ref, w1_ref, hb_ref,
               z_ref):
    e0 = _elu(r0_ref[...] + bh0_ref[...])
    e1 = _elu(r1_ref[...] + bh1_ref[...])
    z_ref[...] = (jnp.dot(e0, w0_ref[...], preferred_element_type=jnp.float32)
                  + jnp.dot(e1, w1_ref[...], preferred_element_type=jnp.float32)
                  + hb_ref[...])


def _node_outs():
    return (
        [jax.ShapeDtypeStruct((NP, DHP), jnp.float32),
         jax.ShapeDtypeStruct((NP, DHP), jnp.float32),
         jax.ShapeDtypeStruct((NP, H), jnp.float32),
         jax.ShapeDtypeStruct((NP, H), jnp.float32),
         jax.ShapeDtypeStruct((1, H), jnp.float32)],
        [pl.BlockSpec((BLK, DHP), lambda i: (i, 0)),
         pl.BlockSpec((BLK, DHP), lambda i: (i, 0)),
         pl.BlockSpec((BLK, H), lambda i: (i, 0)),
         pl.BlockSpec((BLK, H), lambda i: (i, 0)),
         pl.BlockSpec((1, H), lambda i: (0, 0))],
    )


def _full(shape):
    return pl.BlockSpec(shape, lambda i: tuple(0 for _ in shape))


def _tc_layer0(x_pad, W, As, Ad):
    outs, ospecs = _node_outs()
    return pl.pallas_call(
        _layer0_body,
        grid=(NP // BLK,),
        in_specs=[pl.BlockSpec((BLK, 128), lambda i: (i, 0)),
                  _full((128, D)), _full((D, H)), _full((D, H))],
        out_specs=ospecs,
        out_shape=outs,
    )(x_pad, W, As, Ad)


def _tc_layerk(r0, r1, bh0, bh1, Wp0, Wp1, As, Ad):
    outs, ospecs = _node_outs()
    return pl.pallas_call(
        _layerk_body,
        grid=(NP // BLK,),
        in_specs=[pl.BlockSpec((BLK, DHP), lambda i: (i, 0)),
                  pl.BlockSpec((BLK, DHP), lambda i: (i, 0)),
                  _full((1, DHP)), _full((1, DHP)),
                  _full((DHP, D)), _full((DHP, D)),
                  _full((D, H)), _full((D, H))],
        out_specs=ospecs,
        out_shape=outs,
    )(r0, r1, bh0, bh1, Wp0, Wp1, As, Ad)


def _tc_head(r0, r1, bh0, bh1, hw0, hw1, hb):
    HB = 200
    return pl.pallas_call(
        _head_body,
        grid=(N // HB,),
        in_specs=[pl.BlockSpec((HB, DHP), lambda i: (i, 0)),
                  pl.BlockSpec((HB, DHP), lambda i: (i, 0)),
                  _full((1, DHP)), _full((1, DHP)),
                  _full((DHP, DH)), _full((DHP, DH)),
                  _full((1, DH))],
        out_specs=pl.BlockSpec((HB, DH), lambda i: (i, 0)),
        out_shape=jax.ShapeDtypeStruct((N, DH), jnp.float32),
    )(r0, r1, bh0, bh1, hw0, hw1, hb)


@functools.lru_cache(maxsize=None)
def _sc_mesh():
    return plsc.VectorSubcoreMesh(core_axis_name="c", subcore_axis_name="s",
                                  num_cores=2, num_subcores=16)


_SC_PARAMS = pltpu.CompilerParams(needs_layout_passes=False,
                                  use_tc_tiling_on_sc=False)


def _iota16():
    return lax.iota(jnp.int32, 16)


def _g16(ref2d, k):
    i = _iota16()
    return plsc.load_gather(ref2d, [(i >> 3) + 2 * k, i & 7])


def _s16(ref2d, k, x):
    i = _iota16()
    plsc.store_scatter(ref2d, [(i >> 3) + 2 * k, i & 7], x)


def _lrelu16(x):
    return jnp.where(x > 0, x, 0.2 * x)


@functools.lru_cache(maxsize=None)
def _make_pass_a():
    buf = lambda shape, dt: [pltpu.VMEM(shape, dt), pltpu.VMEM(shape, dt)]
    return pl.kernel(
        _sc_pass_a,
        out_type=[jax.ShapeDtypeStruct((EP, H), jnp.float32),
                  jax.ShapeDtypeStruct((NP, H), jnp.float32),
                  jax.ShapeDtypeStruct((NP, H), jnp.float32)],
        mesh=_sc_mesh(),
        compiler_params=_SC_PARAMS,
        scratch_types=[buf((CHA,), jnp.int32),
                       buf((CHA,), jnp.int32),
                       buf((CHA, H), jnp.float32),
                       buf((CHA, H), jnp.float32),
                       pltpu.VMEM((CHA, H), jnp.float32),
                       pltpu.VMEM((16,), jnp.float32),
                       pltpu.VMEM((RPT, H), jnp.float32),
                       pltpu.VMEM_SHARED((NP, H), jnp.float32),
                       [pltpu.SemaphoreType.DMA, pltpu.SemaphoreType.DMA],
                       [pltpu.SemaphoreType.DMA, pltpu.SemaphoreType.DMA]],
    )


def _sc_pass_a(src_hbm, dst_hbm, es_hbm, ed_hbm, mg_hbm,
               ex_hbm, s0_hbm, s1_hbm,
               src_v, dst_v, esg, edg, exv, mgv, zbuf, s_sh, sem1, sem2):
    c = lax.axis_index("c")
    s = lax.axis_index("s")
    base = (c * 16 + s) * (EP // NTILES)

    def zb(k, _):
        _s16(zbuf, k, jnp.zeros((16,), jnp.float32))
        return 0
    lax.fori_loop(0, RPT * H // 16, zb, 0)
    pltpu.sync_copy(zbuf, s_sh.at[pl.ds(s * RPT, RPT)])
    pltpu.sync_copy(mg_hbm, mgv)
    plsc.subcore_barrier()

    mgvec = mgv[...]

    def start(i, b):
        off = base + i * CHA
        pltpu.sync_copy(src_hbm.at[pl.ds(off, CHA)], src_v[b])
        pltpu.sync_copy(dst_hbm.at[pl.ds(off, CHA)], dst_v[b])
        pltpu.async_copy(es_hbm.at[src_v[b]], esg[b], sem1[b])
        pltpu.async_copy(ed_hbm.at[dst_v[b]], edg[b], sem2[b])

    def process(i, b):
        off = base + i * CHA
        pltpu.make_async_copy(es_hbm.at[src_v[b]], esg[b], sem1[b]).wait()
        pltpu.make_async_copy(ed_hbm.at[dst_v[b]], edg[b], sem2[b]).wait()

        @plsc.parallel_loop(0, CHA // 2, step=1, unroll=4)
        def inner(k):
            esv = _g16(esg[b], k)
            edv = _g16(edg[b], k)
            e = _lrelu16(esv + edv)
            bb = _lrelu16(mgvec + edv)
            _s16(exv, k, jnp.exp(e - bb))
        pltpu.sync_copy(exv, ex_hbm.at[pl.ds(off, CHA)])
        pltpu.sync_copy(exv, s_sh.at[dst_v[b]], add=True)

    start(0, 0)

    def pair(g, _):
        start(2 * g + 1, 1)
        process(2 * g, 0)
        start(2 * g + 2, 0)
        process(2 * g + 1, 1)
        return 0
    lax.fori_loop(0, ITA // 2, pair, 0)
    process(ITA - 1, 0)

    plsc.subcore_barrier()
    rs = pl.ds(s * RPT, RPT)

    @pl.when(c == 0)
    def _():
        pltpu.sync_copy(s_sh.at[rs], s0_hbm.at[rs])

    @pl.when(c == 1)
    def _():
        pltpu.sync_copy(s_sh.at[rs], s1_hbm.at[rs])


@functools.lru_cache(maxsize=None)
def _make_pass_b():
    buf = lambda shape, dt: [pltpu.VMEM(shape, dt) for _ in range(3)]
    sem3 = lambda: [pltpu.SemaphoreType.DMA for _ in range(3)]
    return pl.kernel(
        _sc_pass_b,
        out_type=[jax.ShapeDtypeStruct((NP, DHP), jnp.float32),
                  jax.ShapeDtypeStruct((NP, DHP), jnp.float32)],
        mesh=_sc_mesh(),
        compiler_params=_SC_PARAMS,
        scratch_types=[buf((CHB,), jnp.int32),
                       buf((CHB,), jnp.int32),
                       buf((CHB, DHP), jnp.float32),
                       buf((CHB, H), jnp.float32),
                       buf((CHB, H), jnp.float32),
                       buf((CHB, H), jnp.float32),
                       pltpu.VMEM_SHARED((NA, DHP), jnp.float32),
                       sem3(), sem3(), sem3(), sem3(), sem3()],
    )


def _sc_pass_b(src_hbm, dst_hbm, ex_hbm, s0_hbm, s1_hbm, hp0_hbm, hp1_hbm,
               r0_hbm, r1_hbm,
               src_v, dst_v, rows_v, exv, s0g, s1g, out_sh,
               sem1, sem2, sem3, sem4, semw):
    c = lax.axis_index("c")
    s = lax.axis_index("s")

    def half(hp_hbm, r_hbm, head_base):
        i16 = _iota16()
        hms = []
        for j in range(DHP // 16):
            pos = i16 + j * 16
            hms.append(jnp.minimum(pos // C, H // 2 - 1) + head_base)

        def zero_rows0():
            def zrow(e, _):
                for j in range(DHP // 16):
                    plsc.store_scatter(
                        rows_v[0], [jnp.broadcast_to(e, (16,)), i16 + j * 16],
                        jnp.zeros((16,), jnp.float32))
                return 0
            lax.fori_loop(0, CHB, zrow, 0)

        zero_rows0()
        for t in range(RPB // CHB):
            pltpu.sync_copy(rows_v[0], out_sh.at[pl.ds(s * RPB + t * CHB, CHB)])
        rem = RPB % CHB
        if rem:
            pltpu.sync_copy(
                rows_v[0].at[pl.ds(0, rem)],
                out_sh.at[pl.ds(s * RPB + (RPB // CHB) * CHB, rem)])
        plsc.subcore_barrier()

        def wait_scatter(b):
            pltpu.make_async_copy(rows_v[b], out_sh.at[dst_v[b]],
                                  semw[b]).wait()

        def start(i, b, w):
            if w:
                wait_scatter(b)
            off = s * EPT_B + i * CHB
            pltpu.sync_copy(src_hbm.at[pl.ds(off, CHB)], src_v[b])
            pltpu.sync_copy(dst_hbm.at[pl.ds(off, CHB)], dst_v[b])
            pltpu.async_copy(hp_hbm.at[src_v[b]], rows_v[b], sem1[b])
            pltpu.async_copy(ex_hbm.at[pl.ds(off, CHB)], exv[b], sem2[b])
            pltpu.async_copy(s0_hbm.at[dst_v[b]], s0g[b], sem3[b])
            pltpu.async_copy(s1_hbm.at[dst_v[b]], s1g[b], sem4[b])

        def process(i, b):
            off = s * EPT_B + i * CHB
            pltpu.make_async_copy(ex_hbm.at[pl.ds(off, CHB)], exv[b],
                                  sem2[b]).wait()
            pltpu.make_async_copy(s0_hbm.at[dst_v[b]], s0g[b], sem3[b]).wait()
            pltpu.make_async_copy(s1_hbm.at[dst_v[b]], s1g[b], sem4[b]).wait()

            @plsc.parallel_loop(0, CHB // 2, step=1, unroll=4)
            def aloop(k):
                ex16 = _g16(exv[b], k)
                sv = _g16(s0g[b], k) + _g16(s1g[b], k)
                _s16(exv[b], k, ex16 / (sv + 1e-16))
            pltpu.make_async_copy(hp_hbm.at[src_v[b]], rows_v[b],
                                  sem1[b]).wait()

            @plsc.parallel_loop(0, CHB, step=1, unroll=2)
            def scale(e):
                erow = jnp.broadcast_to(e, (16,))
                for j in range(DHP // 16):
                    cols = i16 + j * 16
                    a = plsc.load_gather(exv[b], [erow, hms[j]])
                    v = plsc.load_gather(rows_v[b], [erow, cols])
                    plsc.store_scatter(rows_v[b], [erow, cols], v * a)
            pltpu.async_copy(rows_v[b], out_sh.at[dst_v[b]], semw[b], add=True)

        start(0, 0, False)
        start(1, 1, False)
        process(0, 0)
        start(2, 2, False)
        process(1, 1)
        start(3, 0, True)
        process(2, 2)
        start(4, 1, True)

        def triple(g, _):
            process(3 * g, 0)
            start(3 * g + 2, 2, True)
            process(3 * g + 1, 1)
            start(3 * g + 3, 0, True)
            process(3 * g + 2, 2)
            start(3 * g + 4, 1, True)
            return 0
        lax.fori_loop(1, ITB // 3 - 1, triple, 0)
        process(ITB - 3, 0)
        start(ITB - 1, 2, True)
        process(ITB - 2, 1)
        process(ITB - 1, 2)
        wait_scatter(0)
        wait_scatter(1)
        wait_scatter(2)

        plsc.subcore_barrier()
        rs = pl.ds(s * RPB, RPB)
        pltpu.sync_copy(out_sh.at[rs], r_hbm.at[rs])

        @pl.when(s == 0)
        def _():
            zero_rows0()
            for t in range((NP - NA) // CHB):
                pltpu.sync_copy(rows_v[0],
                                r_hbm.at[pl.ds(NA + t * CHB, CHB)])
            rem2 = (NP - NA) % CHB
            if rem2:
                pltpu.sync_copy(
                    rows_v[0].at[pl.ds(0, rem2)],
                    r_hbm.at[pl.ds(NA + ((NP - NA) // CHB) * CHB, rem2)])

    @pl.when(c == 0)
    def _():
        half(hp0_hbm, r0_hbm, 0)

    @pl.when(c == 1)
    def _():
        half(hp1_hbm, r1_hbm, H // 2)


def _split_pad_W(W):
    Wp0 = jnp.pad(W[:DH], ((0, DHP - DH), (0, 0)))
    Wp1 = jnp.pad(W[DH:], ((0, DHP - DH), (0, 0)))
    return Wp0, Wp1


def _att_mat(a):
    rows = jnp.arange(D)
    return jnp.zeros((D, H), jnp.float32).at[rows, rows // C].set(a.reshape(-1))


def _bias_halves(b):
    bh0 = jnp.pad(b[:DH], (0, DHP - DH)).reshape(1, DHP)
    bh1 = jnp.pad(b[DH:], (0, DHP - DH)).reshape(1, DHP)
    return bh0, bh1


def kernel(x, edge_index, W0, as0, ad0, b0, W1, as1, ad1, b1,
           W2, as2, ad2, b2, W3, as3, ad3, b3, head_W, head_b):
    loop = jnp.arange(N, dtype=edge_index.dtype)
    pad = jnp.full((EP - ET,), N, dtype=edge_index.dtype)
    srcT = jnp.concatenate([edge_index[0], loop, pad])
    dstT = jnp.concatenate([edge_index[1], loop, pad])

    x_pad = jnp.pad(x, ((0, NP - N), (0, 0)))
    params = [(W0, as0, ad0, b0), (W1, as1, ad1, b1),
              (W2, as2, ad2, b2), (W3, as3, ad3, b3)]

    pass_a = _make_pass_a()
    pass_b = _make_pass_b()

    hp0, hp1, es, ed, mg = _tc_layer0(x_pad, W0, _att_mat(as0), _att_mat(ad0))
    for k in (1, 2, 3):
        mg16 = jnp.concatenate([mg[0], mg[0]])
        ex, s0, s1 = pass_a(srcT, dstT, es, ed, mg16)
        r0, r1 = pass_b(srcT, dstT, ex, s0, s1, hp0, hp1)
        W, a_s, a_d, b_prev = params[k][0], params[k][1], params[k][2], params[k - 1][3]
        bh0, bh1 = _bias_halves(b_prev)
        Wp0, Wp1 = _split_pad_W(W)
        hp0, hp1, es, ed, mg = _tc_layerk(r0, r1, bh0, bh1, Wp0, Wp1,
                                          _att_mat(a_s), _att_mat(a_d))
    mg16 = jnp.concatenate([mg[0], mg[0]])
    ex, s0, s1 = pass_a(srcT, dstT, es, ed, mg16)
    r0, r1 = pass_b(srcT, dstT, ex, s0, s1, hp0, hp1)

    bh0, bh1 = _bias_halves(b3)
    hw0 = jnp.pad(head_W[:DH], ((0, DHP - DH), (0, 0)))
    hw1 = jnp.pad(head_W[DH:], ((0, DHP - DH), (0, 0)))
    return _tc_head(r0, r1, bh0, bh1, hw0, hw1, head_b.reshape(1, DH))


# speedup vs baseline: 1.1476x; 1.0082x over previous
import functools

import jax
import jax.numpy as jnp
from jax import lax
from jax.experimental import pallas as pl
from jax.experimental.pallas import tpu as pltpu
from jax.experimental.pallas import tpu_sc as plsc

H = 8
C = 33
D = H * C
DH = 132
DHP = 144
BLK = 256
NP = 10240
N = 10000
ET = 330000
EP = 330240
CHA = 240
CHB = 80
NTILES = 32
ITA = EP // NTILES // CHA
EPT_B = EP // 16
ITB = EPT_B // CHB
RPT = NP // 16
NA = 10016
RPB = NA // 16


def _elu(x):
    return jnp.where(x > 0, x, jnp.exp(x) - 1.0)


def _lrelu(x):
    return jnp.where(x > 0, x, 0.2 * x)


def _layer0_body(x_ref, w_ref, as_ref, ad_ref,
                 hp0_ref, hp1_ref, es_ref, ed_ref, mg_ref):
    i = pl.program_id(0)
    hp = jnp.dot(x_ref[...], w_ref[...], preferred_element_type=jnp.float32)
    es = jnp.dot(hp, as_ref[...], preferred_element_type=jnp.float32)
    ed = jnp.dot(hp, ad_ref[...], preferred_element_type=jnp.float32)
    es_ref[...] = es
    ed_ref[...] = ed
    z = jnp.zeros((hp.shape[0], DHP - DH), jnp.float32)
    hp0_ref[...] = jnp.concatenate([hp[:, :DH], z], axis=1)
    hp1_ref[...] = jnp.concatenate([hp[:, DH:], z], axis=1)
    cur = jnp.max(es, axis=0, keepdims=True)

    @pl.when(i == 0)
    def _():
        mg_ref[...] = cur

    @pl.when(i > 0)
    def _():
        mg_ref[...] = jnp.maximum(mg_ref[...], cur)


def _layerk_body(r0_ref, r1_ref, bh0_ref, bh1_ref, w0_ref, w1_ref,
                 as_ref, ad_ref,
                 hp0_ref, hp1_ref, es_ref, ed_ref, mg_ref):
    i = pl.program_id(0)
    e0 = _elu(r0_ref[...] + bh0_ref[...])
    e1 = _elu(r1_ref[...] + bh1_ref[...])
    hp = (jnp.dot(e0, w0_ref[...], preferred_element_type=jnp.float32)
          + jnp.dot(e1, w1_ref[...], preferred_element_type=jnp.float32))
    es = jnp.dot(hp, as_ref[...], preferred_element_type=jnp.float32)
    ed = jnp.dot(hp, ad_ref[...], preferred_element_type=jnp.float32)
    es_ref[...] = es
    ed_ref[...] = ed
    z = jnp.zeros((hp.shape[0], DHP - DH), jnp.float32)
    hp0_ref[...] = jnp.concatenate([hp[:, :DH], z], axis=1)
    hp1_ref[...] = jnp.concatenate([hp[:, DH:], z], axis=1)
    cur = jnp.max(es, axis=0, keepdims=True)

    @pl.when(i == 0)
    def _():
        mg_ref[...] = cur

    @pl.when(i > 0)
    def _():
        mg_ref[...] = jnp.maximum(mg_ref[...], cur)


def _head_body(r0_ref, r1_ref, bh0_ref, bh1_ref, w0_ref, w1_ref, hb_ref,
               z_ref):
    e0 = _elu(r0_ref[...] + bh0_ref[...])
    e1 = _elu(r1_ref[...] + bh1_ref[...])
    z_ref[...] = (jnp.dot(e0, w0_ref[...], preferred_element_type=jnp.float32)
                  + jnp.dot(e1, w1_ref[...], preferred_element_type=jnp.float32)
                  + hb_ref[...])


def _node_outs():
    return (
        [jax.ShapeDtypeStruct((NP, DHP), jnp.float32),
         jax.ShapeDtypeStruct((NP, DHP), jnp.float32),
         jax.ShapeDtypeStruct((NP, H), jnp.float32),
         jax.ShapeDtypeStruct((NP, H), jnp.float32),
         jax.ShapeDtypeStruct((1, H), jnp.float32)],
        [pl.BlockSpec((BLK, DHP), lambda i: (i, 0)),
         pl.BlockSpec((BLK, DHP), lambda i: (i, 0)),
         pl.BlockSpec((BLK, H), lambda i: (i, 0)),
         pl.BlockSpec((BLK, H), lambda i: (i, 0)),
         pl.BlockSpec((1, H), lambda i: (0, 0))],
    )


def _full(shape):
    return pl.BlockSpec(shape, lambda i: tuple(0 for _ in shape))


def _tc_layer0(x_pad, W, As, Ad):
    outs, ospecs = _node_outs()
    return pl.pallas_call(
        _layer0_body,
        grid=(NP // BLK,),
        in_specs=[pl.BlockSpec((BLK, 128), lambda i: (i, 0)),
                  _full((128, D)), _full((D, H)), _full((D, H))],
        out_specs=ospecs,
        out_shape=outs,
    )(x_pad, W, As, Ad)


def _tc_layerk(r0, r1, bh0, bh1, Wp0, Wp1, As, Ad):
    outs, ospecs = _node_outs()
    return pl.pallas_call(
        _layerk_body,
        grid=(NP // BLK,),
        in_specs=[pl.BlockSpec((BLK, DHP), lambda i: (i, 0)),
                  pl.BlockSpec((BLK, DHP), lambda i: (i, 0)),
                  _full((1, DHP)), _full((1, DHP)),
                  _full((DHP, D)), _full((DHP, D)),
                  _full((D, H)), _full((D, H))],
        out_specs=ospecs,
        out_shape=outs,
    )(r0, r1, bh0, bh1, Wp0, Wp1, As, Ad)


def _tc_head(r0, r1, bh0, bh1, hw0, hw1, hb):
    HB = 200
    return pl.pallas_call(
        _head_body,
        grid=(N // HB,),
        in_specs=[pl.BlockSpec((HB, DHP), lambda i: (i, 0)),
                  pl.BlockSpec((HB, DHP), lambda i: (i, 0)),
                  _full((1, DHP)), _full((1, DHP)),
                  _full((DHP, DH)), _full((DHP, DH)),
                  _full((1, DH))],
        out_specs=pl.BlockSpec((HB, DH), lambda i: (i, 0)),
        out_shape=jax.ShapeDtypeStruct((N, DH), jnp.float32),
    )(r0, r1, bh0, bh1, hw0, hw1, hb)


@functools.lru_cache(maxsize=None)
def _sc_mesh():
    return plsc.VectorSubcoreMesh(core_axis_name="c", subcore_axis_name="s",
                                  num_cores=2, num_subcores=16)


_SC_PARAMS = pltpu.CompilerParams(needs_layout_passes=False,
                                  use_tc_tiling_on_sc=False)


def _iota16():
    return lax.iota(jnp.int32, 16)


def _g16(ref2d, k):
    i = _iota16()
    return plsc.load_gather(ref2d, [(i >> 3) + 2 * k, i & 7])


def _s16(ref2d, k, x):
    i = _iota16()
    plsc.store_scatter(ref2d, [(i >> 3) + 2 * k, i & 7], x)


def _lrelu16(x):
    return jnp.where(x > 0, x, 0.2 * x)


@functools.lru_cache(maxsize=None)
def _make_pass_a():
    buf = lambda shape, dt: [pltpu.VMEM(shape, dt) for _ in range(3)]
    sem3 = lambda: [pltpu.SemaphoreType.DMA for _ in range(3)]
    return pl.kernel(
        _sc_pass_a,
        out_type=[jax.ShapeDtypeStruct((EP, H), jnp.float32),
                  jax.ShapeDtypeStruct((NP, H), jnp.float32),
                  jax.ShapeDtypeStruct((NP, H), jnp.float32)],
        mesh=_sc_mesh(),
        compiler_params=_SC_PARAMS,
        scratch_types=[buf((CHA,), jnp.int32),
                       buf((CHA,), jnp.int32),
                       buf((CHA, H), jnp.float32),
                       buf((CHA, H), jnp.float32),
                       buf((CHA, H), jnp.float32),
                       pltpu.VMEM((16,), jnp.float32),
                       pltpu.VMEM((RPT, H), jnp.float32),
                       pltpu.VMEM_SHARED((NP, H), jnp.float32),
                       sem3(), sem3(), sem3(), sem3()],
    )


def _sc_pass_a(src_hbm, dst_hbm, es_hbm, ed_hbm, mg_hbm,
               ex_hbm, s0_hbm, s1_hbm,
               src_v, dst_v, esg, edg, exv, mgv, zbuf, s_sh,
               sem1, sem2, semx, semw):
    c = lax.axis_index("c")
    s = lax.axis_index("s")
    base = (c * 16 + s) * (EP // NTILES)

    def zb(k, _):
        _s16(zbuf, k, jnp.zeros((16,), jnp.float32))
        return 0
    lax.fori_loop(0, RPT * H // 16, zb, 0)
    pltpu.sync_copy(zbuf, s_sh.at[pl.ds(s * RPT, RPT)])
    pltpu.sync_copy(mg_hbm, mgv)
    plsc.subcore_barrier()

    mgvec = mgv[...]

    def drain_out(b):
        pltpu.make_async_copy(exv[b], ex_hbm.at[pl.ds(base, CHA)],
                              semx[b]).wait()
        pltpu.make_async_copy(exv[b], s_sh.at[dst_v[b]], semw[b]).wait()

    def start(i, b, w):
        if w:
            drain_out(b)
        off = base + i * CHA
        pltpu.sync_copy(src_hbm.at[pl.ds(off, CHA)], src_v[b])
        pltpu.sync_copy(dst_hbm.at[pl.ds(off, CHA)], dst_v[b])
        pltpu.async_copy(es_hbm.at[src_v[b]], esg[b], sem1[b])
        pltpu.async_copy(ed_hbm.at[dst_v[b]], edg[b], sem2[b])

    def process(i, b):
        off = base + i * CHA
        pltpu.make_async_copy(es_hbm.at[src_v[b]], esg[b], sem1[b]).wait()
        pltpu.make_async_copy(ed_hbm.at[dst_v[b]], edg[b], sem2[b]).wait()

        @plsc.parallel_loop(0, CHA // 2, step=1, unroll=4)
        def inner(k):
            esv = _g16(esg[b], k)
            edv = _g16(edg[b], k)
            e = _lrelu16(esv + edv)
            bb = _lrelu16(mgvec + edv)
            _s16(exv[b], k, jnp.exp(e - bb))
        pltpu.async_copy(exv[b], ex_hbm.at[pl.ds(off, CHA)], semx[b])
        pltpu.async_copy(exv[b], s_sh.at[dst_v[b]], semw[b], add=True)

    start(0, 0, False)
    start(1, 1, False)
    process(0, 0)
    start(2, 2, False)
    process(1, 1)
    start(3, 0, True)
    process(2, 2)
    start(4, 1, True)

    def triple(g, _):
        process(3 * g, 0)
        start(3 * g + 2, 2, True)
        process(3 * g + 1, 1)
        start(3 * g + 3, 0, True)
        process(3 * g + 2, 2)
        start(3 * g + 4, 1, True)
        return 0
    lax.fori_loop(1, 13, triple, 0)
    process(39, 0)
    start(41, 2, True)
    process(40, 1)
    start(42, 0, True)
    process(41, 2)
    process(42, 0)
    drain_out(1)
    drain_out(2)
    drain_out(0)

    plsc.subcore_barrier()
    rs = pl.ds(s * RPT, RPT)

    @pl.when(c == 0)
    def _():
        pltpu.sync_copy(s_sh.at[rs], s0_hbm.at[rs])

    @pl.when(c == 1)
    def _():
        pltpu.sync_copy(s_sh.at[rs], s1_hbm.at[rs])


@functools.lru_cache(maxsize=None)
def _make_pass_b():
    buf = lambda shape, dt: [pltpu.VMEM(shape, dt) for _ in range(3)]
    sem3 = lambda: [pltpu.SemaphoreType.DMA for _ in range(3)]
    return pl.kernel(
        _sc_pass_b,
        out_type=[jax.ShapeDtypeStruct((NP, DHP), jnp.float32),
                  jax.ShapeDtypeStruct((NP, DHP), jnp.float32)],
        mesh=_sc_mesh(),
        compiler_params=_SC_PARAMS,
        scratch_types=[buf((CHB,), jnp.int32),
                       buf((CHB,), jnp.int32),
                       buf((CHB, DHP), jnp.float32),
                       buf((CHB, H), jnp.float32),
                       buf((CHB, H), jnp.float32),
                       buf((CHB, H), jnp.float32),
                       pltpu.VMEM_SHARED((NA, DHP), jnp.float32),
                       sem3(), sem3(), sem3(), sem3(), sem3()],
    )


def _sc_pass_b(src_hbm, dst_hbm, ex_hbm, s0_hbm, s1_hbm, hp0_hbm, hp1_hbm,
               r0_hbm, r1_hbm,
               src_v, dst_v, rows_v, exv, s0g, s1g, out_sh,
               sem1, sem2, sem3, sem4, semw):
    c = lax.axis_index("c")
    s = lax.axis_index("s")

    def half(hp_hbm, r_hbm, head_base):
        i16 = _iota16()
        hms = []
        for j in range(DHP // 16):
            pos = i16 + j * 16
            hms.append(jnp.minimum(pos // C, H // 2 - 1) + head_base)

        def zero_rows0():
            def zrow(e, _):
                for j in range(DHP // 16):
                    plsc.store_scatter(
                        rows_v[0], [jnp.broadcast_to(e, (16,)), i16 + j * 16],
                        jnp.zeros((16,), jnp.float32))
                return 0
            lax.fori_loop(0, CHB, zrow, 0)

        zero_rows0()
        for t in range(RPB // CHB):
            pltpu.sync_copy(rows_v[0], out_sh.at[pl.ds(s * RPB + t * CHB, CHB)])
        rem = RPB % CHB
        if rem:
            pltpu.sync_copy(
                rows_v[0].at[pl.ds(0, rem)],
                out_sh.at[pl.ds(s * RPB + (RPB // CHB) * CHB, rem)])
        plsc.subcore_barrier()

        def wait_scatter(b):
            pltpu.make_async_copy(rows_v[b], out_sh.at[dst_v[b]],
                                  semw[b]).wait()

        def start(i, b, w):
            if w:
                wait_scatter(b)
            off = s * EPT_B + i * CHB
            pltpu.sync_copy(src_hbm.at[pl.ds(off, CHB)], src_v[b])
            pltpu.sync_copy(dst_hbm.at[pl.ds(off, CHB)], dst_v[b])
            pltpu.async_copy(hp_hbm.at[src_v[b]], rows_v[b], sem1[b])
            pltpu.async_copy(ex_hbm.at[pl.ds(off, CHB)], exv[b], sem2[b])
            pltpu.async_copy(s0_hbm.at[dst_v[b]], s0g[b], sem3[b])
            pltpu.async_copy(s1_hbm.at[dst_v[b]], s1g[b], sem4[b])

        def process(i, b):
            off = s * EPT_B + i * CHB
            pltpu.make_async_copy(ex_hbm.at[pl.ds(off, CHB)], exv[b],
                                  sem2[b]).wait()
            pltpu.make_async_copy(s0_hbm.at[dst_v[b]], s0g[b], sem3[b]).wait()
            pltpu.make_async_copy(s1_hbm.at[dst_v[b]], s1g[b], sem4[b]).wait()

            @plsc.parallel_loop(0, CHB // 2, step=1, unroll=4)
            def aloop(k):
                ex16 = _g16(exv[b], k)
                sv = _g16(s0g[b], k) + _g16(s1g[b], k)
                _s16(exv[b], k, ex16 / (sv + 1e-16))
            pltpu.make_async_copy(hp_hbm.at[src_v[b]], rows_v[b],
                                  sem1[b]).wait()

            @plsc.parallel_loop(0, CHB, step=1, unroll=2)
            def scale(e):
                erow = jnp.broadcast_to(e, (16,))
                for j in range(DHP // 16):
                    cols = i16 + j * 16
                    a = plsc.load_gather(exv[b], [erow, hms[j]])
                    v = plsc.load_gather(rows_v[b], [erow, cols])
                    plsc.store_scatter(rows_v[b], [erow, cols], v * a)
            pltpu.async_copy(rows_v[b], out_sh.at[dst_v[b]], semw[b], add=True)

        start(0, 0, False)
        start(1, 1, False)
        process(0, 0)
        start(2, 2, False)
        process(1, 1)
        start(3, 0, True)
        process(2, 2)
        start(4, 1, True)

        def triple(g, _):
            process(3 * g, 0)
            start(3 * g + 2, 2, True)
            process(3 * g + 1, 1)
            start(3 * g + 3, 0, True)
            process(3 * g + 2, 2)
            start(3 * g + 4, 1, True)
            return 0
        lax.fori_loop(1, ITB // 3 - 1, triple, 0)
        process(ITB - 3, 0)
        start(ITB - 1, 2, True)
        process(ITB - 2, 1)
        process(ITB - 1, 2)
        wait_scatter(0)
        wait_scatter(1)
        wait_scatter(2)

        plsc.subcore_barrier()
        rs = pl.ds(s * RPB, RPB)
        pltpu.sync_copy(out_sh.at[rs], r_hbm.at[rs])

        @pl.when(s == 0)
        def _():
            zero_rows0()
            for t in range((NP - NA) // CHB):
                pltpu.sync_copy(rows_v[0],
                                r_hbm.at[pl.ds(NA + t * CHB, CHB)])
            rem2 = (NP - NA) % CHB
            if rem2:
                pltpu.sync_copy(
                    rows_v[0].at[pl.ds(0, rem2)],
                    r_hbm.at[pl.ds(NA + ((NP - NA) // CHB) * CHB, rem2)])

    @pl.when(c == 0)
    def _():
        half(hp0_hbm, r0_hbm, 0)

    @pl.when(c == 1)
    def _():
        half(hp1_hbm, r1_hbm, H // 2)


def _split_pad_W(W):
    Wp0 = jnp.pad(W[:DH], ((0, DHP - DH), (0, 0)))
    Wp1 = jnp.pad(W[DH:], ((0, DHP - DH), (0, 0)))
    return Wp0, Wp1


def _att_mat(a):
    rows = jnp.arange(D)
    return jnp.zeros((D, H), jnp.float32).at[rows, rows // C].set(a.reshape(-1))


def _bias_halves(b):
    bh0 = jnp.pad(b[:DH], (0, DHP - DH)).reshape(1, DHP)
    bh1 = jnp.pad(b[DH:], (0, DHP - DH)).reshape(1, DHP)
    return bh0, bh1


def kernel(x, edge_index, W0, as0, ad0, b0, W1, as1, ad1, b1,
           W2, as2, ad2, b2, W3, as3, ad3, b3, head_W, head_b):
    loop = jnp.arange(N, dtype=edge_index.dtype)
    pad = jnp.full((EP - ET,), N, dtype=edge_index.dtype)
    srcT = jnp.concatenate([edge_index[0], loop, pad])
    dstT = jnp.concatenate([edge_index[1], loop, pad])

    x_pad = jnp.pad(x, ((0, NP - N), (0, 0)))
    params = [(W0, as0, ad0, b0), (W1, as1, ad1, b1),
              (W2, as2, ad2, b2), (W3, as3, ad3, b3)]

    pass_a = _make_pass_a()
    pass_b = _make_pass_b()

    hp0, hp1, es, ed, mg = _tc_layer0(x_pad, W0, _att_mat(as0), _att_mat(ad0))
    for k in (1, 2, 3):
        mg16 = jnp.concatenate([mg[0], mg[0]])
        ex, s0, s1 = pass_a(srcT, dstT, es, ed, mg16)
        r0, r1 = pass_b(srcT, dstT, ex, s0, s1, hp0, hp1)
        W, a_s, a_d, b_prev = params[k][0], params[k][1], params[k][2], params[k - 1][3]
        bh0, bh1 = _bias_halves(b_prev)
        Wp0, Wp1 = _split_pad_W(W)
        hp0, hp1, es, ed, mg = _tc_layerk(r0, r1, bh0, bh1, Wp0, Wp1,
                                          _att_mat(a_s), _att_mat(a_d))
    mg16 = jnp.concatenate([mg[0], mg[0]])
    ex, s0, s1 = pass_a(srcT, dstT, es, ed, mg16)
    r0, r1 = pass_b(srcT, dstT, ex, s0, s1, hp0, hp1)

    bh0, bh1 = _bias_halves(b3)
    hw0 = jnp.pad(head_W[:DH], ((0, DHP - DH), (0, 0)))
    hw1 = jnp.pad(head_W[DH:], ((0, DHP - DH), (0, 0)))
    return _tc_head(r0, r1, bh0, bh1, hw0, hw1, head_b.reshape(1, DH))
